# Initial kernel scaffold; baseline (speedup 1.0000x reference)
import jax
import jax.numpy as jnp
from jax.experimental import pallas as pl


def kernel(inp, edge_index, edge_vals, weights):
    raise NotImplementedError("write your pallas kernel here")


# speedup vs baseline: 3.5933x; 3.5933x over previous
import functools
import jax
import jax.numpy as jnp
from jax import lax
from jax.experimental import pallas as pl
from jax.experimental.pallas import tpu as pltpu
from jax.experimental.pallas import tpu_sc as plsc

NC = 2
NS = 16
L = 16
NW = NC * NS
B = 128

_GATHER_DNUMS = lax.GatherDimensionNumbers(
    offset_dims=(), collapsed_slice_dims=(0,), start_index_map=(0,))


def _bcast_lane(vec, i):
    idx = jnp.full((L, 1), i, jnp.int32)
    return lax.gather(vec, idx, _GATHER_DNUMS, slice_sizes=(1,),
                      mode=lax.GatherScatterMode.PROMISE_IN_BOUNDS)


def _matmul(inp, weights):
    n, d_in = inp.shape
    r, _, d_out = weights.shape
    br = 2000

    def body(x_ref, w_ref, y_ref):
        y_ref[0] = jnp.dot(x_ref[...], w_ref[0],
                           preferred_element_type=jnp.float32)

    return pl.pallas_call(
        body,
        grid=(r, n // br),
        in_specs=[
            pl.BlockSpec((br, d_in), lambda ri, i: (i, 0)),
            pl.BlockSpec((1, d_in, d_out), lambda ri, i: (ri, 0, 0)),
        ],
        out_specs=pl.BlockSpec((1, br, d_out), lambda ri, i: (ri, i, 0)),
        out_shape=jax.ShapeDtypeStruct((r, n, d_out), jnp.float32),
    )(inp, weights)


def _add_partials(partials, n):
    d = partials.shape[2]
    br = 1000

    def body(p_ref, o_ref):
        o_ref[...] = p_ref[0] + p_ref[1]

    return pl.pallas_call(
        body,
        grid=(n // br,),
        in_specs=[pl.BlockSpec((2, br, d), lambda i: (0, i, 0))],
        out_specs=pl.BlockSpec((br, d), lambda i: (i, 0)),
        out_shape=jax.ShapeDtypeStruct((n, d), jnp.float32),
    )(partials)


def _sc_edge_kernel(n, d, pe):
    cpw = pe // NW
    ch = cpw // B
    npad = -(-n // (NS * B)) * (NS * B)
    zr = npad // NS
    mesh = plsc.VectorSubcoreMesh(core_axis_name="c", subcore_axis_name="s",
                                  num_cores=NC, num_subcores=NS)

    @functools.partial(
        pl.kernel,
        out_type=jax.ShapeDtypeStruct((NC, npad, d), jnp.float32),
        mesh=mesh,
        scratch_types=[
            pltpu.VMEM((B,), jnp.int32),
            pltpu.VMEM((B,), jnp.int32),
            pltpu.VMEM((B,), jnp.float32),
            pltpu.VMEM((B, d), jnp.float32),
            pltpu.VMEM_SHARED((npad, d), jnp.float32),
            pltpu.SemaphoreType.DMA,
        ],
    )
    def k(y_hbm, src_hbm, dst_hbm, val_hbm, out_hbm,
          src_v, dst_v, val_v, rows_v, acc, sem):
        c = lax.axis_index("c")
        s = lax.axis_index("s")
        wid = c * NS + s

        @pl.loop(0, B)
        def _(i):
            for j in range(d // L):
                rows_v[i, pl.ds(j * L, L)] = jnp.zeros((L,), jnp.float32)

        @pl.loop(0, zr // B)
        def _(q):
            pltpu.sync_copy(rows_v, acc.at[pl.ds(s * zr + q * B, B), :])

        plsc.subcore_barrier()

        base = wid * cpw

        @pl.loop(0, ch)
        def _(it):
            off = base + it * B
            pltpu.sync_copy(src_hbm.at[pl.ds(off, B)], src_v)
            gather = pltpu.async_copy(y_hbm.at[src_v], rows_v, sem)
            pltpu.sync_copy(val_hbm.at[pl.ds(off, B)], val_v)
            pltpu.sync_copy(dst_hbm.at[pl.ds(off, B)], dst_v)
            gather.wait()

            @pl.loop(0, B // L)
            def _(g):
                vvec = val_v[pl.ds(g * L, L)]
                for i in range(L):
                    vb = _bcast_lane(vvec, i)
                    for j in range(d // L):
                        sl = rows_v[g * L + i, pl.ds(j * L, L)]
                        rows_v[g * L + i, pl.ds(j * L, L)] = sl * vb

            pltpu.sync_copy(rows_v, acc.at[dst_v], add=True)

        plsc.subcore_barrier()
        pltpu.sync_copy(acc.at[pl.ds(s * zr, zr), :],
                        out_hbm.at[c, pl.ds(s * zr, zr), :])

    return k


def kernel(inp, edge_index, edge_vals, weights):
    n, d_in = inp.shape
    r, _, d_out = weights.shape
    e = edge_index.shape[2]

    y = _matmul(inp, weights).reshape(r * n, d_out)

    rel_off = (jnp.arange(r, dtype=jnp.int32) * n)[:, None]
    src = (edge_index[:, 1, :].astype(jnp.int32) + rel_off).reshape(-1)
    dst = edge_index[:, 0, :].astype(jnp.int32).reshape(-1)
    vals = edge_vals.reshape(-1).astype(jnp.float32)

    te = r * e
    pe = -(-te // (NW * B)) * (NW * B)
    pad = pe - te
    if pad:
        src = jnp.concatenate([src, jnp.zeros((pad,), jnp.int32)])
        dst = jnp.concatenate([dst, jnp.zeros((pad,), jnp.int32)])
        vals = jnp.concatenate([vals, jnp.zeros((pad,), jnp.float32)])

    partials = _sc_edge_kernel(n, d_out, pe)(y, src, dst, vals)
    return _add_partials(partials, n)
